# Initial kernel scaffold; baseline (speedup 1.0000x reference)
import jax
import jax.numpy as jnp
from jax.experimental import pallas as pl


def kernel(x, edge_index, W, b):
    raise NotImplementedError("write your pallas kernel here")


# speedup vs baseline: 17.3250x; 17.3250x over previous
import functools
import math

import jax
import jax.numpy as jnp
from jax import lax
from jax.experimental import pallas as pl
from jax.experimental.pallas import tpu as pltpu
from jax.experimental.pallas import tpu_sc as plsc

N = 10000
E = 320000
D = 128

NC = 2
NS = 16
L = 16
NW = NC * NS
C = 128
EPW = E // NW
CH = math.ceil(EPW / C)
EPW_PAD = CH * C
E_PAD = EPW_PAD * NW
N_PAD = 10240
RPT = N_PAD // NS
DEG_W = 16

_MESH = plsc.VectorSubcoreMesh(core_axis_name="c", subcore_axis_name="s")


@functools.partial(
    pl.kernel,
    out_type=jax.ShapeDtypeStruct((NC, N_PAD, DEG_W), jnp.float32),
    mesh=_MESH,
    scratch_types=[
        pltpu.VMEM((CH, C), jnp.int32),
        pltpu.VMEM((C, DEG_W), jnp.float32),
        pltpu.VMEM_SHARED((N_PAD, DEG_W), jnp.float32),
    ],
)
def _deg_kernel(coli_hbm, zeros_hbm, out_hbm, col_v, ones_v, dacc_sh):
    cid = lax.axis_index("c")
    sid = lax.axis_index("s")
    wid = sid * NC + cid

    e0 = jnp.where(lax.iota(jnp.int32, L) == 0,
                   jnp.full((L,), 1.0, jnp.float32),
                   jnp.full((L,), 0.0, jnp.float32))

    def _fill(i, carry):
        ones_v[i] = e0
        return carry
    lax.fori_loop(0, C, _fill, 0)

    pltpu.sync_copy(zeros_hbm.at[pl.ds(sid * RPT, RPT)],
                    dacc_sh.at[pl.ds(sid * RPT, RPT)])
    pltpu.sync_copy(coli_hbm.at[wid], col_v)
    plsc.subcore_barrier()

    def _body(j, carry):
        pltpu.sync_copy(ones_v, dacc_sh.at[col_v.at[j]], add=True)
        return carry
    lax.fori_loop(0, CH, _body, 0)

    plsc.subcore_barrier()
    pltpu.sync_copy(dacc_sh.at[pl.ds(sid * RPT, RPT)],
                    out_hbm.at[cid, pl.ds(sid * RPT, RPT)])


@functools.partial(
    pl.kernel,
    out_type=jax.ShapeDtypeStruct((NC, N_PAD, D), jnp.float32),
    mesh=_MESH,
    scratch_types=[
        pltpu.VMEM((CH, C), jnp.int32),
        pltpu.VMEM((CH, C), jnp.int32),
        pltpu.VMEM((C, D), jnp.float32),
        pltpu.VMEM_SHARED((N_PAD, D), jnp.float32),
        pltpu.SemaphoreType.DMA,
    ],
)
def _agg_kernel(g_hbm, rowi_hbm, coli_hbm, zeros_hbm, out_hbm,
                row_v, col_v, rows_v, acc_sh, gsem):
    cid = lax.axis_index("c")
    sid = lax.axis_index("s")
    wid = sid * NC + cid

    pltpu.sync_copy(zeros_hbm.at[pl.ds(sid * RPT, RPT)],
                    acc_sh.at[pl.ds(sid * RPT, RPT)])
    pltpu.sync_copy(rowi_hbm.at[wid], row_v)
    pltpu.sync_copy(coli_hbm.at[wid], col_v)
    plsc.subcore_barrier()

    def _body(j, carry):
        pltpu.async_copy(g_hbm.at[row_v.at[j]], rows_v, gsem).wait()
        pltpu.sync_copy(rows_v, acc_sh.at[col_v.at[j]], add=True)
        return carry
    lax.fori_loop(0, CH, _body, 0)

    plsc.subcore_barrier()
    pltpu.sync_copy(acc_sh.at[pl.ds(sid * RPT, RPT)],
                    out_hbm.at[cid, pl.ds(sid * RPT, RPT)])


BM = 1000


def _mm_body(x_ref, w_ref, d0_ref, d1_ref, o_ref):
    deg = d0_ref[:, 0:1] + d1_ref[:, 0:1] + 1.0
    dinv = lax.rsqrt(deg)
    o_ref[...] = jnp.dot(x_ref[...], w_ref[...],
                         preferred_element_type=jnp.float32) * dinv


def _final_body(a0_ref, a1_ref, g_ref, d0_ref, d1_ref, b_ref, o_ref):
    deg = d0_ref[:, 0:1] + d1_ref[:, 0:1] + 1.0
    dinv = lax.rsqrt(deg)
    s = dinv * (a0_ref[...] + a1_ref[...] + g_ref[...]) + b_ref[...]
    o_ref[...] = jnp.maximum(s, 0.0)


def kernel(x, edge_index, W, b):
    ei = edge_index.astype(jnp.int32)
    row = ei[0]
    col = ei[1]
    pad = E_PAD - E
    rowp = jnp.concatenate([row, jnp.zeros((pad,), jnp.int32)]).reshape(NW, CH, C)
    colp = jnp.concatenate([col, jnp.full((pad,), N, jnp.int32)]).reshape(NW, CH, C)

    zeros_deg = jnp.zeros((N_PAD, DEG_W), jnp.float32)
    zeros_acc = jnp.zeros((N_PAD, D), jnp.float32)

    degp = _deg_kernel(colp, zeros_deg)

    g = pl.pallas_call(
        _mm_body,
        grid=(N // BM,),
        in_specs=[
            pl.BlockSpec((BM, D), lambda i: (i, 0)),
            pl.BlockSpec((D, D), lambda i: (0, 0)),
            pl.BlockSpec((BM, DEG_W), lambda i: (i, 0)),
            pl.BlockSpec((BM, DEG_W), lambda i: (i, 0)),
        ],
        out_specs=pl.BlockSpec((BM, D), lambda i: (i, 0)),
        out_shape=jax.ShapeDtypeStruct((N, D), jnp.float32),
    )(x, W, degp[0], degp[1])

    acc = _agg_kernel(g, rowp, colp, zeros_acc)

    out = pl.pallas_call(
        _final_body,
        grid=(N // BM,),
        in_specs=[
            pl.BlockSpec((BM, D), lambda i: (i, 0)),
            pl.BlockSpec((BM, D), lambda i: (i, 0)),
            pl.BlockSpec((BM, D), lambda i: (i, 0)),
            pl.BlockSpec((BM, DEG_W), lambda i: (i, 0)),
            pl.BlockSpec((BM, DEG_W), lambda i: (i, 0)),
            pl.BlockSpec((1, D), lambda i: (0, 0)),
        ],
        out_specs=pl.BlockSpec((BM, D), lambda i: (i, 0)),
        out_shape=jax.ShapeDtypeStruct((N, D), jnp.float32),
    )(acc[0], acc[1], g, degp[0], degp[1], b.reshape(1, D))

    return out
